# speedup vs baseline: 2.2512x; 2.2512x over previous
import functools

import jax
import jax.numpy as jnp
from jax import lax
from jax.experimental import pallas as pl
from jax.experimental.pallas import tpu as pltpu

K = 1024
C = 64
HW = 1024


def _vq_kernel(xb_ref, cb_ref, codes_ref, idx_ref):
    xb = xb_ref[0]
    cb = cb_ref[...]

    mm = lax.dot_general(cb, xb, (((1,), (0,)), ((), ())),
                         preferred_element_type=jnp.float32)
    cb_sqr = jnp.sum(cb * cb, axis=1, keepdims=True)
    x_sqr = jnp.sum(xb * xb, axis=0, keepdims=True)
    dist = cb_sqr + x_sqr - 2.0 * mm

    minval = jnp.min(dist, axis=0, keepdims=True)
    iota_k = lax.broadcasted_iota(jnp.int32, (K, HW), 0)
    masked = jnp.where(dist == minval, iota_k, K)
    idx = jnp.min(masked, axis=0, keepdims=True)
    idx_ref[0] = idx

    onehot = (iota_k == idx).astype(jnp.float32)
    codes = lax.dot_general(cb, onehot, (((0,), (0,)), ((), ())),
                            preferred_element_type=jnp.float32)
    codes_ref[0] = codes


@jax.jit
def kernel(x, codebook):
    B = x.shape[0]
    xf = x.reshape(B, C, HW)
    codes, idx = pl.pallas_call(
        _vq_kernel,
        grid=(B,),
        in_specs=[
            pl.BlockSpec((1, C, HW), lambda b: (b, 0, 0)),
            pl.BlockSpec((K, C), lambda b: (0, 0)),
        ],
        out_specs=[
            pl.BlockSpec((1, C, HW), lambda b: (b, 0, 0)),
            pl.BlockSpec((1, 1, HW), lambda b: (b, 0, 0)),
        ],
        out_shape=[
            jax.ShapeDtypeStruct((B, C, HW), jnp.float32),
            jax.ShapeDtypeStruct((B, 1, HW), jnp.int32),
        ],
    )(xf, codebook)
    codes_out = codes.reshape(B, C, 32, 32)
    ind_out = idx.reshape(B, 32, 32)
    return codes_out, ind_out


# speedup vs baseline: 2.2836x; 1.0144x over previous
import functools

import jax
import jax.numpy as jnp
from jax import lax
from jax.experimental import pallas as pl
from jax.experimental.pallas import tpu as pltpu

K = 1024
C = 64
HW = 1024


def _vq_kernel(xb_ref, cb_ref, codes_ref, idx_ref):
    xb = xb_ref[0]
    cb = cb_ref[...]

    mm = lax.dot_general(cb, xb, (((1,), (0,)), ((), ())),
                         preferred_element_type=jnp.float32)
    cb_sqr = jnp.sum(cb * cb, axis=1, keepdims=True)
    x_sqr = jnp.sum(xb * xb, axis=0, keepdims=True)
    dist = cb_sqr + x_sqr - 2.0 * mm

    minval = jnp.min(dist, axis=0, keepdims=True)
    iota_f = lax.broadcasted_iota(jnp.int32, (K, HW), 0).astype(jnp.float32)
    masked = jnp.where(dist == minval, iota_f, jnp.float32(K))
    idx_f = jnp.min(masked, axis=0, keepdims=True)
    idx_ref[0] = idx_f.astype(jnp.int32)

    onehot = jnp.where(masked == idx_f, 1.0, 0.0)
    codes = lax.dot_general(cb, onehot, (((0,), (0,)), ((), ())),
                            preferred_element_type=jnp.float32)
    codes_ref[0] = codes


@jax.jit
def kernel(x, codebook):
    B = x.shape[0]
    xf = x.reshape(B, C, HW)
    codes, idx = pl.pallas_call(
        _vq_kernel,
        grid=(B,),
        in_specs=[
            pl.BlockSpec((1, C, HW), lambda b: (b, 0, 0)),
            pl.BlockSpec((K, C), lambda b: (0, 0)),
        ],
        out_specs=[
            pl.BlockSpec((1, C, HW), lambda b: (b, 0, 0)),
            pl.BlockSpec((1, 1, HW), lambda b: (b, 0, 0)),
        ],
        out_shape=[
            jax.ShapeDtypeStruct((B, C, HW), jnp.float32),
            jax.ShapeDtypeStruct((B, 1, HW), jnp.int32),
        ],
    )(xf, codebook)
    codes_out = codes.reshape(B, C, 32, 32)
    ind_out = idx.reshape(B, 32, 32)
    return codes_out, ind_out
